# Initial kernel scaffold; baseline (speedup 1.0000x reference)
import jax
import jax.numpy as jnp
from jax.experimental import pallas as pl


def kernel(Decoder_Hidden_State, Batch_decoder_input_hh, Batch_Decoder_Input_Attender, sparse_indices, sparse_val, W_hh, W_map):
    raise NotImplementedError("write your pallas kernel here")


# speedup vs baseline: 2.4729x; 2.4729x over previous
import functools

import jax
import jax.numpy as jnp
from jax import lax
from jax.experimental import pallas as pl
from jax.experimental.pallas import tpu as pltpu
from jax.experimental.pallas import tpu_sc as plsc

_LANES = 16


def _sc_build_mask(flat_idx, B, S, NNZ):
    mesh = plsc.VectorSubcoreMesh(core_axis_name="c", subcore_axis_name="s")

    @functools.partial(
        pl.kernel,
        mesh=mesh,
        out_type=jax.ShapeDtypeStruct((B * S,), jnp.float32),
        scratch_types=[
            pltpu.VMEM((2 * NNZ,), jnp.int32),
            pltpu.VMEM((B * S,), jnp.float32),
        ],
        compiler_params=pltpu.CompilerParams(needs_layout_passes=False),
    )
    def k(idx_hbm, out_hbm, idx_v, table_v):
        cid = lax.axis_index("c")
        sid = lax.axis_index("s")

        @pl.when(jnp.logical_and(cid == 0, sid == 0))
        def _():
            pltpu.sync_copy(idx_hbm, idx_v)
            zeros = jnp.zeros((_LANES,), jnp.float32)
            ones = jnp.ones((_LANES,), jnp.float32)

            def zero_body(i, carry):
                table_v[pl.ds(i * _LANES, _LANES)] = zeros
                return carry

            lax.fori_loop(0, (B * S) // _LANES, zero_body, 0)

            def scat_body(i, carry):
                bvec = idx_v[pl.ds(i * _LANES, _LANES)]
                svec = idx_v[pl.ds(NNZ + i * _LANES, _LANES)]
                plsc.store_scatter(table_v, [bvec * S + svec], ones)
                return carry

            lax.fori_loop(0, NNZ // _LANES, scat_body, 0)
            pltpu.sync_copy(table_v, out_hbm)

    return k(flat_idx)


def _tc_addvals(dhs16, w_hh, w_row, factor):
    V, H = w_hh.shape
    VT = 512
    nv = V // VT

    def body(dhs_ref, whh_ref, w_ref, f_ref, out_ref):
        i = pl.program_id(0)
        dec = lax.dot_general(
            dhs_ref[...], whh_ref[...], (((1,), (1,)), ((), ())),
            preferred_element_type=jnp.float32)
        part = jnp.sum(dec * w_ref[...], axis=1)

        @pl.when(i == 0)
        def _():
            out_ref[...] = jnp.zeros_like(out_ref)

        out_ref[...] += (f_ref[0, 0] * part)[:, None]

    return pl.pallas_call(
        body,
        grid=(nv,),
        in_specs=[
            pl.BlockSpec((16, H), lambda i: (0, 0)),
            pl.BlockSpec((VT, H), lambda i: (i, 0)),
            pl.BlockSpec((1, VT), lambda i: (0, i)),
            pl.BlockSpec((1, 1), lambda i: (0, 0)),
        ],
        out_specs=pl.BlockSpec((16, 1), lambda i: (0, 0)),
        out_shape=jax.ShapeDtypeStruct((16, 1), jnp.float32),
    )(dhs16, w_hh, w_row, factor)


def _tc_alpha(bd_hh, w_row, mask_r, addv, s_t):
    B, S, V = bd_hh.shape
    nj = S // s_t

    def body(bd_ref, w_ref, add_ref, av_ref, alpha_ref):
        b = pl.program_id(0)
        j = pl.program_id(1)
        base = lax.dot_general(
            w_ref[...], bd_ref[0], (((1,), (1,)), ((), ())),
            preferred_element_type=jnp.float32)
        alpha_ref[0, :, pl.ds(j * s_t, s_t)] = base + add_ref[0] * av_ref[b]

        @pl.when(j == nj - 1)
        def _():
            r = alpha_ref[0, :, :]
            msk = r != 0.0
            m = jnp.max(jnp.where(msk, r, -jnp.inf))
            m = jnp.where(m == -jnp.inf, 0.0, m)
            e = jnp.where(msk, jnp.exp(r - m), 0.0)
            den = jnp.sum(e)
            alpha_ref[0, :, :] = e / jnp.where(den == 0.0, 1.0, den)

    return pl.pallas_call(
        body,
        grid=(B, nj),
        in_specs=[
            pl.BlockSpec((1, s_t, V), lambda b, j: (b, j, 0)),
            pl.BlockSpec((1, V), lambda b, j: (0, 0)),
            pl.BlockSpec((1, 1, s_t), lambda b, j: (b * nj + j, 0, 0)),
            pl.BlockSpec(memory_space=pltpu.SMEM),
        ],
        out_specs=pl.BlockSpec((1, 1, S), lambda b, j: (b, 0, 0)),
        out_shape=jax.ShapeDtypeStruct((B, 1, S), jnp.float32),
    )(bd_hh, w_row, mask_r, addv)


def _tc_context(alpha3, att, s_t):
    B, S, V = att.shape
    nj = S // s_t

    def body(alpha_ref, att_ref, c_ref):
        j = pl.program_id(1)
        a = alpha_ref[0, :, pl.ds(j * s_t, s_t)]
        part = lax.dot_general(
            a, att_ref[0], (((1,), (0,)), ((), ())),
            preferred_element_type=jnp.float32)

        @pl.when(j == 0)
        def _():
            c_ref[0] = part

        @pl.when(j > 0)
        def _():
            c_ref[0] += part

    return pl.pallas_call(
        body,
        grid=(B, nj),
        in_specs=[
            pl.BlockSpec((1, 1, S), lambda b, j: (b, 0, 0)),
            pl.BlockSpec((1, s_t, V), lambda b, j: (b, j, 0)),
        ],
        out_specs=pl.BlockSpec((1, 1, V), lambda b, j: (b, 0, 0)),
        out_shape=jax.ShapeDtypeStruct((B, 1, V), jnp.float32),
    )(alpha3, att)


def kernel(Decoder_Hidden_State, Batch_decoder_input_hh, Batch_Decoder_Input_Attender, sparse_indices, sparse_val, W_hh, W_map):
    B, S, V = Batch_decoder_input_hh.shape
    H = Decoder_Hidden_State.shape[1]
    NNZ = sparse_indices.shape[1]
    s_t = 512
    nj = S // s_t

    factor = jnp.where(jnp.asarray(sparse_val) == 1, 1.0, 2.0)
    factor = factor.astype(jnp.float32).reshape(1, 1)
    dhs16 = jnp.pad(Decoder_Hidden_State, ((0, _LANES - B), (0, 0)))
    addv = _tc_addvals(dhs16, W_hh, W_map, factor).reshape(_LANES)

    flat_idx = sparse_indices.astype(jnp.int32).reshape(-1)
    mask = _sc_build_mask(flat_idx, B, S, NNZ)
    mask_r = mask.reshape(B * nj, 1, s_t)

    alpha3 = _tc_alpha(Batch_decoder_input_hh, W_map, mask_r, addv, s_t)
    c3 = _tc_context(alpha3, Batch_Decoder_Input_Attender, s_t)
    return c3.reshape(B, V), alpha3.reshape(B, S, 1)
